# baseline (speedup 1.0000x reference)
import functools

import jax
import jax.numpy as jnp
from jax import lax
from jax.experimental import pallas as pl
from jax.experimental.pallas import tpu as pltpu
from jax.experimental.pallas import tpu_sc as plsc

_MAXG = 5976
_V = _MAXG + 1
_T, _B, _R, _C, _D, _NINV = 16, 32, 21, 79, 64, 55
_TB = _T * _B
_RC = _R * _C
_DG = 8
_NDG = _D // _DG
_NPG = 32 // _NDG
_TBP = _TB // _NPG
_GW = 16
_VT = 376


def _body(g3_hbm, tabp_hbm, inv3_hbm, y_hbm, x_hbm,
          scr_hbm, vic_hbm, invo_hbm,
          tbl_v, ybuf, xbuf, vbuf, ibuf,
          gbuf0, gbuf1, invb0, invb1, st0, st1,
          semg0, semg1, semi0, semi1, semo0, semo1):
    cid = lax.axis_index("c")
    sid = lax.axis_index("s")
    wid = sid * 2 + cid
    dg = wid % _NDG
    pg = wid // _NDG
    d0 = pl.multiple_of(dg * _DG, _DG)
    tb0 = pl.multiple_of(pg * _TBP, _TBP)

    iota = lax.broadcasted_iota(jnp.int32, (16,), 0)
    dvecs = [jnp.full((16,), dd, jnp.int32) for dd in range(_DG)]
    zvec = jnp.zeros((16,), jnp.int32)
    civ = [iota + k * 16 for k in range(5)]
    m_last = iota < (_C - 64)

    ri = iota // 3
    ci = iota - ri * 3
    vic_lane = iota < 9

    pltpu.sync_copy(tabp_hbm.at[dg], tbl_v)
    pltpu.sync_copy(y_hbm.at[pl.ds(tb0, _TBP)], ybuf)
    pltpu.sync_copy(x_hbm.at[pl.ds(tb0, _TBP)], xbuf)

    bufs = ((gbuf0, invb0, st0, semg0, semi0, semo0),
            (gbuf1, invb1, st1, semg1, semi1, semo1))
    for b in range(2):
        gbuf, invb, _, semg, semi, _ = bufs[b]
        pltpu.async_copy(g3_hbm.at[tb0 + b], gbuf, semg)
        pltpu.async_copy(inv3_hbm.at[tb0 + b], invb, semi)

    def t2_body(t2, _):
        for b in range(2):
            gbuf, invb, st, semg, semi, semo = bufs[b]
            tl = t2 * 2 + b
            t = tb0 + tl
            tt = t // _B
            bb = t % _B

            pltpu.make_async_copy(g3_hbm.at[t], gbuf, semg).wait()
            pltpu.make_async_copy(inv3_hbm.at[t], invb, semi).wait()

            @pl.when(t2 > 0)
            def _wait_out():
                tp = t - 2
                pltpu.make_async_copy(
                    st, scr_hbm.at[tp // _B, tp % _B, pl.ds(d0, _DG), :, :],
                    semo).wait()

            def r_body(r):
                rvec = jnp.full((16,), r, jnp.int32)
                r79 = r * _C
                for k in range(5):
                    p = civ[k] + r79
                    gid = plsc.load_gather(gbuf, [p >> 7, p & 127])
                    grow = (gid >> 7) << 3
                    glane = gid & 127
                    m = m_last if k == 4 else None
                    for dd in range(_DG):
                        val = plsc.load_gather(tbl_v, [grow + dd, glane])
                        plsc.store_scatter(st, [dvecs[dd], rvec, civ[k]],
                                           val, mask=m)

            plsc.parallel_loop(0, _R, unroll=2)(r_body)

            tvec = jnp.full((16,), tl, jnp.int32)
            yv = jnp.clip(plsc.load_gather(ybuf, [tvec]), 0, _R - 1)
            xv = jnp.clip(plsc.load_gather(xbuf, [tvec]), 0, _C - 1)
            row = yv - 1 + ri
            col = xv - 1 + ci
            valid = ((row >= 0) & (row < _R) & (col >= 0) & (col < _C)
                     & vic_lane)
            flat = jnp.clip(row * _C + col, 0, _RC - 1)
            g9 = jnp.where(valid,
                           plsc.load_gather(gbuf, [flat >> 7, flat & 127]),
                           _MAXG)
            g9row = (g9 >> 7) << 3
            g9lane = g9 & 127
            tv63 = jnp.full((16,), tl & 63, jnp.int32)
            for dd in range(_DG):
                vals = plsc.load_gather(tbl_v, [g9row + dd, g9lane])
                plsc.store_scatter(vbuf, [tv63, iota + dd * 9], vals,
                                   mask=vic_lane)

            tv7 = jnp.full((16,), tl & 7, jnp.int32)
            for jj in range(4):
                gi = plsc.load_gather(invb, [zvec, iota + jj * 16])
                girow = (gi >> 7) << 3
                gilane = gi & 127
                m = (iota + jj * 16) < _NINV
                for dd in range(_DG):
                    vals = plsc.load_gather(tbl_v, [girow + dd, gilane])
                    plsc.store_scatter(ibuf, [dvecs[dd], tv7,
                                              iota + jj * 16], vals, mask=m)

            @pl.when(tl + 2 < _TBP)
            def _prefetch():
                pltpu.async_copy(g3_hbm.at[t + 2], gbuf, semg)
                pltpu.async_copy(inv3_hbm.at[t + 2], invb, semi)

            pltpu.async_copy(
                st, scr_hbm.at[tt, bb, pl.ds(d0, _DG), :, :], semo)

            @pl.when((tl & 7) == 7)
            def _flush_inv():
                t8 = pl.multiple_of(tb0 + (tl & ~7), 8)
                pltpu.sync_copy(ibuf, invo_hbm.at[dg, :, pl.ds(t8, 8), :])

            @pl.when((tl & 63) == 63)
            def _flush_vic():
                t64 = pl.multiple_of(tb0 + (tl & ~63), 64)
                pltpu.sync_copy(vbuf, vic_hbm.at[dg, pl.ds(t64, 64), :])
        return 0

    lax.fori_loop(0, _TBP // 2, t2_body, 0, unroll=False)

    for b in range(2):
        _, _, st, _, _, semo = bufs[b]
        t = tb0 + _TBP - 2 + b
        pltpu.make_async_copy(
            st, scr_hbm.at[t // _B, t % _B, pl.ds(d0, _DG), :, :],
            semo).wait()


@jax.jit
def _sc_call(g3, tabp, inv3, y, x):
    mesh = plsc.VectorSubcoreMesh(core_axis_name="c", subcore_axis_name="s")
    fn = pl.kernel(
        _body,
        out_type=(
            jax.ShapeDtypeStruct((_T, _B, _D, 24, 128), jnp.float32),
            jax.ShapeDtypeStruct((_NDG, _TB, 128), jnp.float32),
            jax.ShapeDtypeStruct((_NDG, _DG, _TB, 128), jnp.float32),
        ),
        mesh=mesh,
        compiler_params=pltpu.CompilerParams(use_tc_tiling_on_sc=True,
                                             needs_layout_passes=False),
        scratch_types=[
            pltpu.VMEM((_VT, 128), jnp.float32),
            pltpu.VMEM((_TBP,), jnp.int32),
            pltpu.VMEM((_TBP,), jnp.int32),
            pltpu.VMEM((64, 128), jnp.float32),
            pltpu.VMEM((_DG, 8, 128), jnp.float32),
            pltpu.VMEM((_GW, 128), jnp.int32),
            pltpu.VMEM((_GW, 128), jnp.int32),
            pltpu.VMEM((8, 128), jnp.int32),
            pltpu.VMEM((8, 128), jnp.int32),
            pltpu.VMEM((_DG, 24, 128), jnp.float32),
            pltpu.VMEM((_DG, 24, 128), jnp.float32),
            pltpu.SemaphoreType.DMA,
            pltpu.SemaphoreType.DMA,
            pltpu.SemaphoreType.DMA,
            pltpu.SemaphoreType.DMA,
            pltpu.SemaphoreType.DMA,
            pltpu.SemaphoreType.DMA,
        ],
    )
    return fn(g3, tabp, inv3, y, x)


def kernel(glyphs, blstats, inv_glyphs, emb_table):
    T, B, R, C = glyphs.shape
    g_flat = glyphs.reshape(T * B, R * C).astype(jnp.int32)
    g3 = jnp.pad(g_flat, ((0, 0), (0, _GW * 128 - _RC))).reshape(_TB, _GW, 128)
    inv = inv_glyphs.reshape(T * B, _NINV).astype(jnp.int32)
    inv3 = jnp.pad(inv, ((0, 0), (0, 1024 - _NINV))).reshape(_TB, 8, 128)
    tabT = jnp.pad(emb_table.T, ((0, 0), (0, 47 * 128 - _V)))
    tabp = (tabT.reshape(_NDG, _DG, 47, 128)
            .transpose(0, 2, 1, 3)
            .reshape(_NDG, _VT, 128))
    y = blstats[..., 1].reshape(-1).astype(jnp.int32)
    x = blstats[..., 0].reshape(-1).astype(jnp.int32)
    scr_pad, vic_o, inv_o = _sc_call(g3, tabp, inv3, y, x)
    screen = scr_pad[:, :, :, :R, :C]
    vicinity = (vic_o[:, :, :_DG * 9].reshape(_NDG, _TB, _DG, 9)
                .transpose(1, 0, 2, 3).reshape(T, B, _D, 3, 3))
    inventory = (inv_o[:, :, :, :_NINV].transpose(2, 0, 1, 3)
                 .reshape(T, B, _D, _NINV))
    self_ = vicinity[..., 1, 1]
    return screen, vicinity, inventory, self_


# speedup vs baseline: 1.5500x; 1.5500x over previous
import functools

import jax
import jax.numpy as jnp
from jax import lax
from jax.experimental import pallas as pl
from jax.experimental.pallas import tpu as pltpu
from jax.experimental.pallas import tpu_sc as plsc

_MAXG = 5976
_V = _MAXG + 1
_T, _B, _R, _C, _D, _NINV = 16, 32, 21, 79, 64, 55
_TB = _T * _B
_RC = _R * _C
_DG = 8
_NDG = _D // _DG
_NPG = 32 // _NDG
_TBP = _TB // _NPG
_GW = 16
_VT = 376


def _body(g3_hbm, tabp_hbm, inv3_hbm, y_hbm, x_hbm,
          scr_hbm, vic_hbm, invo_hbm,
          tbl_v, ybuf, xbuf, vbuf, ibuf,
          gbuf0, gbuf1, invb0, invb1, st0, st1,
          semg0, semg1, semi0, semi1, semo0, semo1):
    cid = lax.axis_index("c")
    sid = lax.axis_index("s")
    wid = sid * 2 + cid
    dg = wid % _NDG
    pg = wid // _NDG
    d0 = pl.multiple_of(dg * _DG, _DG)
    tb0 = pl.multiple_of(pg * _TBP, _TBP)

    iota = lax.broadcasted_iota(jnp.int32, (16,), 0)
    dvecs = [jnp.full((16,), dd, jnp.int32) for dd in range(_DG)]
    zvec = jnp.zeros((16,), jnp.int32)
    civ = [iota + k * 16 for k in range(5)]
    m_last = iota < (_C - 64)

    ri = iota // 3
    ci = iota - ri * 3
    vic_lane = iota < 9

    pltpu.sync_copy(tabp_hbm.at[dg], tbl_v)
    pltpu.sync_copy(y_hbm.at[pl.ds(tb0, _TBP)], ybuf)
    pltpu.sync_copy(x_hbm.at[pl.ds(tb0, _TBP)], xbuf)

    bufs = ((gbuf0, invb0, st0, semg0, semi0, semo0),
            (gbuf1, invb1, st1, semg1, semi1, semo1))
    for b in range(2):
        gbuf, invb, _, semg, semi, _ = bufs[b]
        pltpu.async_copy(g3_hbm.at[tb0 + b], gbuf, semg)
        pltpu.async_copy(inv3_hbm.at[tb0 + b], invb, semi)

    def t2_body(t2, _):
        for b in range(2):
            gbuf, invb, st, semg, semi, semo = bufs[b]
            tl = t2 * 2 + b
            t = tb0 + tl
            tt = t // _B
            bb = t % _B

            pltpu.make_async_copy(g3_hbm.at[t], gbuf, semg).wait()
            pltpu.make_async_copy(inv3_hbm.at[t], invb, semi).wait()

            @pl.when(t2 > 0)
            def _wait_out():
                tp = t - 2
                pltpu.make_async_copy(
                    st, scr_hbm.at[tp // _B, tp % _B, pl.ds(d0, _DG), :, :],
                    semo).wait()

            def r_body(r):
                rvec = jnp.full((16,), r, jnp.int32)
                r79 = r * _C
                for k in range(5):
                    p = civ[k] + r79
                    gid = plsc.load_gather(gbuf, [p >> 7, p & 127])
                    grow = (gid >> 7) << 3
                    glane = gid & 127
                    m = m_last if k == 4 else None
                    for dd in range(_DG):
                        val = plsc.load_gather(tbl_v, [grow + dd, glane])
                        plsc.store_scatter(st, [dvecs[dd], rvec, civ[k]],
                                           val, mask=m)

            plsc.parallel_loop(0, _R)(r_body)

            tvec = jnp.full((16,), tl, jnp.int32)
            yv = jnp.clip(plsc.load_gather(ybuf, [tvec]), 0, _R - 1)
            xv = jnp.clip(plsc.load_gather(xbuf, [tvec]), 0, _C - 1)
            row = yv - 1 + ri
            col = xv - 1 + ci
            valid = ((row >= 0) & (row < _R) & (col >= 0) & (col < _C)
                     & vic_lane)
            flat = jnp.clip(row * _C + col, 0, _RC - 1)
            g9 = jnp.where(valid,
                           plsc.load_gather(gbuf, [flat >> 7, flat & 127]),
                           _MAXG)
            g9row = (g9 >> 7) << 3
            g9lane = g9 & 127
            tv63 = jnp.full((16,), tl & 63, jnp.int32)
            for dd in range(_DG):
                vals = plsc.load_gather(tbl_v, [g9row + dd, g9lane])
                plsc.store_scatter(vbuf, [tv63, iota + dd * 9], vals,
                                   mask=vic_lane)

            tv7 = jnp.full((16,), tl & 7, jnp.int32)
            for jj in range(4):
                gi = plsc.load_gather(invb, [zvec, iota + jj * 16])
                girow = (gi >> 7) << 3
                gilane = gi & 127
                m = (iota + jj * 16) < _NINV
                for dd in range(_DG):
                    vals = plsc.load_gather(tbl_v, [girow + dd, gilane])
                    plsc.store_scatter(ibuf, [dvecs[dd], tv7,
                                              iota + jj * 16], vals, mask=m)

            @pl.when(tl + 2 < _TBP)
            def _prefetch():
                pltpu.async_copy(g3_hbm.at[t + 2], gbuf, semg)
                pltpu.async_copy(inv3_hbm.at[t + 2], invb, semi)

            pltpu.async_copy(
                st, scr_hbm.at[tt, bb, pl.ds(d0, _DG), :, :], semo)

            @pl.when((tl & 7) == 7)
            def _flush_inv():
                t8 = pl.multiple_of(tb0 + (tl & ~7), 8)
                pltpu.sync_copy(ibuf, invo_hbm.at[dg, :, pl.ds(t8, 8), :])

            @pl.when((tl & 63) == 63)
            def _flush_vic():
                t64 = pl.multiple_of(tb0 + (tl & ~63), 64)
                pltpu.sync_copy(vbuf, vic_hbm.at[dg, pl.ds(t64, 64), :])
        return 0

    lax.fori_loop(0, _TBP // 2, t2_body, 0, unroll=False)

    for b in range(2):
        _, _, st, _, _, semo = bufs[b]
        t = tb0 + _TBP - 2 + b
        pltpu.make_async_copy(
            st, scr_hbm.at[t // _B, t % _B, pl.ds(d0, _DG), :, :],
            semo).wait()


@jax.jit
def _sc_call(g3, tabp, inv3, y, x):
    mesh = plsc.VectorSubcoreMesh(core_axis_name="c", subcore_axis_name="s")
    fn = pl.kernel(
        _body,
        out_type=(
            jax.ShapeDtypeStruct((_T, _B, _D, 24, 128), jnp.float32),
            jax.ShapeDtypeStruct((_NDG, _TB, 128), jnp.float32),
            jax.ShapeDtypeStruct((_NDG, _DG, _TB, 128), jnp.float32),
        ),
        mesh=mesh,
        compiler_params=pltpu.CompilerParams(use_tc_tiling_on_sc=True,
                                             needs_layout_passes=False),
        scratch_types=[
            pltpu.VMEM((_VT, 128), jnp.float32),
            pltpu.VMEM((_TBP,), jnp.int32),
            pltpu.VMEM((_TBP,), jnp.int32),
            pltpu.VMEM((64, 128), jnp.float32),
            pltpu.VMEM((_DG, 8, 128), jnp.float32),
            pltpu.VMEM((_GW, 128), jnp.int32),
            pltpu.VMEM((_GW, 128), jnp.int32),
            pltpu.VMEM((8, 128), jnp.int32),
            pltpu.VMEM((8, 128), jnp.int32),
            pltpu.VMEM((_DG, 24, 128), jnp.float32),
            pltpu.VMEM((_DG, 24, 128), jnp.float32),
            pltpu.SemaphoreType.DMA,
            pltpu.SemaphoreType.DMA,
            pltpu.SemaphoreType.DMA,
            pltpu.SemaphoreType.DMA,
            pltpu.SemaphoreType.DMA,
            pltpu.SemaphoreType.DMA,
        ],
    )
    return fn(g3, tabp, inv3, y, x)


def kernel(glyphs, blstats, inv_glyphs, emb_table):
    T, B, R, C = glyphs.shape
    g_flat = glyphs.reshape(T * B, R * C).astype(jnp.int32)
    g3 = jnp.pad(g_flat, ((0, 0), (0, _GW * 128 - _RC))).reshape(_TB, _GW, 128)
    inv = inv_glyphs.reshape(T * B, _NINV).astype(jnp.int32)
    inv3 = jnp.pad(inv, ((0, 0), (0, 1024 - _NINV))).reshape(_TB, 8, 128)
    tabT = jnp.pad(emb_table.T, ((0, 0), (0, 47 * 128 - _V)))
    tabp = (tabT.reshape(_NDG, _DG, 47, 128)
            .transpose(0, 2, 1, 3)
            .reshape(_NDG, _VT, 128))
    y = blstats[..., 1].reshape(-1).astype(jnp.int32)
    x = blstats[..., 0].reshape(-1).astype(jnp.int32)
    scr_pad, vic_o, inv_o = _sc_call(g3, tabp, inv3, y, x)
    screen = scr_pad[:, :, :, :R, :C]
    vicinity = (vic_o[:, :, :_DG * 9].reshape(_NDG, _TB, _DG, 9)
                .transpose(1, 0, 2, 3).reshape(T, B, _D, 3, 3))
    inventory = (inv_o[:, :, :, :_NINV].transpose(2, 0, 1, 3)
                 .reshape(T, B, _D, _NINV))
    self_ = vicinity[..., 1, 1]
    return screen, vicinity, inventory, self_


# speedup vs baseline: 1.6436x; 1.0604x over previous
import functools

import jax
import jax.numpy as jnp
from jax import lax
from jax.experimental import pallas as pl
from jax.experimental.pallas import tpu as pltpu
from jax.experimental.pallas import tpu_sc as plsc

_MAXG = 5976
_V = _MAXG + 1
_T, _B, _R, _C, _D, _NINV = 16, 32, 21, 79, 64, 55
_TB = _T * _B
_RC = _R * _C
_DG = 8
_NDG = _D // _DG
_NPG = 32 // _NDG
_TBP = _TB // _NPG
_GW = 16
_VT = 376


def _body(g3_hbm, tabp_hbm, inv3_hbm, y_hbm, x_hbm,
          scr_hbm, vic_hbm, invo_hbm,
          tbl_v, ybuf, xbuf, vbuf, ibuf,
          gbuf0, gbuf1, invb0, invb1, st0, st1,
          semg0, semg1, semi0, semi1, semo0, semo1):
    cid = lax.axis_index("c")
    sid = lax.axis_index("s")
    wid = sid * 2 + cid
    dg = wid % _NDG
    pg = wid // _NDG
    d0 = pl.multiple_of(dg * _DG, _DG)
    tb0 = pl.multiple_of(pg * _TBP, _TBP)

    iota = lax.broadcasted_iota(jnp.int32, (16,), 0)
    dvecs = [jnp.full((16,), dd, jnp.int32) for dd in range(_DG)]
    zvec = jnp.zeros((16,), jnp.int32)
    civ = [iota + k * 16 for k in range(5)]
    m_last = iota < (_C - 64)

    ri = iota // 3
    ci = iota - ri * 3
    vic_lane = iota < 9

    pltpu.sync_copy(tabp_hbm.at[dg], tbl_v)
    pltpu.sync_copy(y_hbm.at[pl.ds(tb0, _TBP)], ybuf)
    pltpu.sync_copy(x_hbm.at[pl.ds(tb0, _TBP)], xbuf)

    bufs = ((gbuf0, invb0, st0, semg0, semi0, semo0),
            (gbuf1, invb1, st1, semg1, semi1, semo1))
    for b in range(2):
        gbuf, invb, _, semg, semi, _ = bufs[b]
        pltpu.async_copy(g3_hbm.at[tb0 + b], gbuf, semg)
        pltpu.async_copy(inv3_hbm.at[tb0 + b], invb, semi)

    def t2_body(t2, _):
        for b in range(2):
            gbuf, invb, st, semg, semi, semo = bufs[b]
            tl = t2 * 2 + b
            t = tb0 + tl
            tt = t // _B
            bb = t % _B

            pltpu.make_async_copy(g3_hbm.at[t], gbuf, semg).wait()
            pltpu.make_async_copy(inv3_hbm.at[t], invb, semi).wait()

            @pl.when(t2 > 0)
            def _wait_out():
                tp = t - 2
                pltpu.make_async_copy(
                    st, scr_hbm.at[tp // _B, tp % _B, :, pl.ds(d0, _DG), :],
                    semo).wait()

            def r_body(r):
                rvec = jnp.full((16,), r, jnp.int32)
                r79 = r * _C
                for k in range(5):
                    p = civ[k] + r79
                    gid = plsc.load_gather(gbuf, [p >> 7, p & 127])
                    grow = (gid >> 7) << 3
                    glane = gid & 127
                    m = m_last if k == 4 else None
                    for dd in range(_DG):
                        val = plsc.load_gather(tbl_v, [grow + dd, glane])
                        plsc.store_scatter(st, [rvec, dvecs[dd], civ[k]],
                                           val, mask=m)

            plsc.parallel_loop(0, _R)(r_body)

            tvec = jnp.full((16,), tl, jnp.int32)
            yv = jnp.clip(plsc.load_gather(ybuf, [tvec]), 0, _R - 1)
            xv = jnp.clip(plsc.load_gather(xbuf, [tvec]), 0, _C - 1)
            row = yv - 1 + ri
            col = xv - 1 + ci
            valid = ((row >= 0) & (row < _R) & (col >= 0) & (col < _C)
                     & vic_lane)
            flat = jnp.clip(row * _C + col, 0, _RC - 1)
            g9 = jnp.where(valid,
                           plsc.load_gather(gbuf, [flat >> 7, flat & 127]),
                           _MAXG)
            g9row = (g9 >> 7) << 3
            g9lane = g9 & 127
            tv63 = jnp.full((16,), tl & 63, jnp.int32)
            for dd in range(_DG):
                vals = plsc.load_gather(tbl_v, [g9row + dd, g9lane])
                plsc.store_scatter(vbuf, [tv63, iota + dd * 9], vals,
                                   mask=vic_lane)

            tv7 = jnp.full((16,), tl & 7, jnp.int32)
            for jj in range(4):
                gi = plsc.load_gather(invb, [zvec, iota + jj * 16])
                girow = (gi >> 7) << 3
                gilane = gi & 127
                m = (iota + jj * 16) < _NINV
                for dd in range(_DG):
                    vals = plsc.load_gather(tbl_v, [girow + dd, gilane])
                    plsc.store_scatter(ibuf, [dvecs[dd], tv7,
                                              iota + jj * 16], vals, mask=m)

            @pl.when(tl + 2 < _TBP)
            def _prefetch():
                pltpu.async_copy(g3_hbm.at[t + 2], gbuf, semg)
                pltpu.async_copy(inv3_hbm.at[t + 2], invb, semi)

            pltpu.async_copy(
                st, scr_hbm.at[tt, bb, :, pl.ds(d0, _DG), :], semo)

            @pl.when((tl & 7) == 7)
            def _flush_inv():
                t8 = pl.multiple_of(tb0 + (tl & ~7), 8)
                pltpu.sync_copy(ibuf, invo_hbm.at[dg, :, pl.ds(t8, 8), :])

            @pl.when((tl & 63) == 63)
            def _flush_vic():
                t64 = pl.multiple_of(tb0 + (tl & ~63), 64)
                pltpu.sync_copy(vbuf, vic_hbm.at[dg, pl.ds(t64, 64), :])
        return 0

    lax.fori_loop(0, _TBP // 2, t2_body, 0, unroll=False)

    for b in range(2):
        _, _, st, _, _, semo = bufs[b]
        t = tb0 + _TBP - 2 + b
        pltpu.make_async_copy(
            st, scr_hbm.at[t // _B, t % _B, :, pl.ds(d0, _DG), :],
            semo).wait()


@jax.jit
def _sc_call(g3, tabp, inv3, y, x):
    mesh = plsc.VectorSubcoreMesh(core_axis_name="c", subcore_axis_name="s")
    fn = pl.kernel(
        _body,
        out_type=(
            jax.ShapeDtypeStruct((_T, _B, _R, _D, 128), jnp.float32),
            jax.ShapeDtypeStruct((_NDG, _TB, 128), jnp.float32),
            jax.ShapeDtypeStruct((_NDG, _DG, _TB, 128), jnp.float32),
        ),
        mesh=mesh,
        compiler_params=pltpu.CompilerParams(use_tc_tiling_on_sc=True,
                                             needs_layout_passes=False),
        scratch_types=[
            pltpu.VMEM((_VT, 128), jnp.float32),
            pltpu.VMEM((_TBP,), jnp.int32),
            pltpu.VMEM((_TBP,), jnp.int32),
            pltpu.VMEM((64, 128), jnp.float32),
            pltpu.VMEM((_DG, 8, 128), jnp.float32),
            pltpu.VMEM((_GW, 128), jnp.int32),
            pltpu.VMEM((_GW, 128), jnp.int32),
            pltpu.VMEM((8, 128), jnp.int32),
            pltpu.VMEM((8, 128), jnp.int32),
            pltpu.VMEM((_R, _DG, 128), jnp.float32),
            pltpu.VMEM((_R, _DG, 128), jnp.float32),
            pltpu.SemaphoreType.DMA,
            pltpu.SemaphoreType.DMA,
            pltpu.SemaphoreType.DMA,
            pltpu.SemaphoreType.DMA,
            pltpu.SemaphoreType.DMA,
            pltpu.SemaphoreType.DMA,
        ],
    )
    return fn(g3, tabp, inv3, y, x)


def kernel(glyphs, blstats, inv_glyphs, emb_table):
    T, B, R, C = glyphs.shape
    g_flat = glyphs.reshape(T * B, R * C).astype(jnp.int32)
    g3 = jnp.pad(g_flat, ((0, 0), (0, _GW * 128 - _RC))).reshape(_TB, _GW, 128)
    inv = inv_glyphs.reshape(T * B, _NINV).astype(jnp.int32)
    inv3 = jnp.pad(inv, ((0, 0), (0, 1024 - _NINV))).reshape(_TB, 8, 128)
    tabT = jnp.pad(emb_table.T, ((0, 0), (0, 47 * 128 - _V)))
    tabp = (tabT.reshape(_NDG, _DG, 47, 128)
            .transpose(0, 2, 1, 3)
            .reshape(_NDG, _VT, 128))
    y = blstats[..., 1].reshape(-1).astype(jnp.int32)
    x = blstats[..., 0].reshape(-1).astype(jnp.int32)
    scr_pad, vic_o, inv_o = _sc_call(g3, tabp, inv3, y, x)
    screen = scr_pad[:, :, :, :, :C].transpose(0, 1, 3, 2, 4)
    vicinity = (vic_o[:, :, :_DG * 9].reshape(_NDG, _TB, _DG, 9)
                .transpose(1, 0, 2, 3).reshape(T, B, _D, 3, 3))
    inventory = (inv_o[:, :, :, :_NINV].transpose(2, 0, 1, 3)
                 .reshape(T, B, _D, _NINV))
    self_ = vicinity[..., 1, 1]
    return screen, vicinity, inventory, self_
